# baseline (speedup 1.0000x reference)
import functools

import jax
import jax.numpy as jnp
from jax import lax
from jax.experimental import pallas as pl
from jax.experimental.pallas import tpu as pltpu
from jax.experimental.pallas import tpu_sc as plsc

N = 10000
E = 320000
D = 128
L = 4
B = 256
MAXF = 16
MAXN = 128
EPS = 1e-5

NP = 10240
BLK = 1024
NWORK = 32
CK = 128
ECH = 2560
GRP = 8
EP = (ECH + GRP) * CK
CPP = ECH // 16
SPLIT0 = 56
SPLIT1 = CPP - SPLIT0
GMAX = -(-max(SPLIT0, SPLIT1) // GRP) * GRP
RPS = NP // 16

_HI = jax.lax.Precision.HIGHEST


def _ln(pre, g, b):
    mu = jnp.mean(pre, axis=-1, keepdims=True)
    d = pre - mu
    var = jnp.mean(d * d, axis=-1, keepdims=True)
    return d * lax.rsqrt(var + EPS) * g + b


def _t0_body(h_ref, wi_ref, wn_ref, t_ref):
    h = h_ref[...]
    t_ref[0] = jnp.dot(h, wi_ref[...], precision=_HI)
    t_ref[1] = jnp.dot(h, wn_ref[...], precision=_HI)


def _t0_call(h, w_inter, w_intra):
    return pl.pallas_call(
        _t0_body,
        grid=(NP // BLK,),
        in_specs=[
            pl.BlockSpec((BLK, D), lambda i: (i, 0)),
            pl.BlockSpec((D, D), lambda i: (0, 0)),
            pl.BlockSpec((D, D), lambda i: (0, 0)),
        ],
        out_specs=pl.BlockSpec((2, BLK, D), lambda i: (0, i, 0)),
        out_shape=jax.ShapeDtypeStruct((2, NP, D), jnp.float32),
    )(h, w_inter, w_intra)


def _upd_body(u_ref, a0_ref, a1_ref, bs_ref, g_ref, b_ref, wi_ref, wn_ref, t_ref):
    pre = u_ref[...] + a0_ref[...] + a1_ref[...] + bs_ref[...]
    h = jnp.maximum(_ln(pre, g_ref[...], b_ref[...]), 0.0)
    t_ref[0] = jnp.dot(h, wi_ref[...], precision=_HI)
    t_ref[1] = jnp.dot(h, wn_ref[...], precision=_HI)


def _upd_call(u, a0, a1, bs, g, b, w_inter, w_intra):
    return pl.pallas_call(
        _upd_body,
        grid=(NP // BLK,),
        in_specs=[
            pl.BlockSpec((BLK, D), lambda i: (i, 0)),
            pl.BlockSpec((BLK, D), lambda i: (i, 0)),
            pl.BlockSpec((BLK, D), lambda i: (i, 0)),
            pl.BlockSpec((1, D), lambda i: (0, 0)),
            pl.BlockSpec((1, D), lambda i: (0, 0)),
            pl.BlockSpec((1, D), lambda i: (0, 0)),
            pl.BlockSpec((D, D), lambda i: (0, 0)),
            pl.BlockSpec((D, D), lambda i: (0, 0)),
        ],
        out_specs=pl.BlockSpec((2, BLK, D), lambda i: (0, i, 0)),
        out_shape=jax.ShapeDtypeStruct((2, NP, D), jnp.float32),
    )(u, a0, a1, bs, g, b, w_inter, w_intra)


def _updlast_body(u_ref, a0_ref, a1_ref, bs_ref, g_ref, b_ref, h_ref):
    pre = u_ref[...] + a0_ref[...] + a1_ref[...] + bs_ref[...]
    h_ref[...] = jnp.maximum(_ln(pre, g_ref[...], b_ref[...]), 0.0)


def _updlast_call(u, a0, a1, bs, g, b):
    return pl.pallas_call(
        _updlast_body,
        grid=(NP // BLK,),
        in_specs=[
            pl.BlockSpec((BLK, D), lambda i: (i, 0)),
            pl.BlockSpec((BLK, D), lambda i: (i, 0)),
            pl.BlockSpec((BLK, D), lambda i: (i, 0)),
            pl.BlockSpec((1, D), lambda i: (0, 0)),
            pl.BlockSpec((1, D), lambda i: (0, 0)),
            pl.BlockSpec((1, D), lambda i: (0, 0)),
        ],
        out_specs=pl.BlockSpec((BLK, D), lambda i: (i, 0)),
        out_shape=jax.ShapeDtypeStruct((NP, D), jnp.float32),
    )(u, a0, a1, bs, g, b)


def _off_body(bv_ref, off_ref, cnt_ref):
    b = pl.program_id(0)
    bv = bv_ref[...]
    off = jnp.sum((bv < b).astype(jnp.int32))
    cnt = jnp.sum((bv == b).astype(jnp.int32))
    off_ref[...] = jnp.full((1, 1, 128), off, jnp.int32)
    cnt_ref[...] = jnp.full((1, 1, 128), cnt, jnp.int32)


def _off_call(bv):
    return pl.pallas_call(
        _off_body,
        grid=(B,),
        in_specs=[pl.BlockSpec((NP // 128, 128), lambda i: (0, 0))],
        out_specs=[
            pl.BlockSpec((1, 1, 128), lambda i: (i, 0, 0)),
            pl.BlockSpec((1, 1, 128), lambda i: (i, 0, 0)),
        ],
        out_shape=[
            jax.ShapeDtypeStruct((B, 1, 128), jnp.int32),
            jax.ShapeDtypeStruct((B, 1, 128), jnp.int32),
        ],
    )(bv)


def _pool_body(h_ref, s_ref, off_ref, cnt_ref, fg_ref, fb_ref, wi_ref, bi_ref,
               fe_ref, fm_ref, reg_ref):
    b = pl.program_id(0)
    off = off_ref[b]
    cnt = cnt_ref[b]
    hs = h_ref[pl.ds(off, MAXN), :]
    ss = s_ref[pl.ds(off, MAXN), :]
    m = (lax.broadcasted_iota(jnp.int32, (MAXN, 1), 0) < cnt).astype(jnp.float32)
    ssm = ss * m
    fe = lax.dot_general(ssm, hs, (((0,), (0,)), ((), ())),
                         precision=_HI, preferred_element_type=jnp.float32)
    fe_ref[...] = _ln(fe, fg_ref[...], fb_ref[...])[None]
    colsum = jnp.sum(ssm, axis=0, keepdims=True)
    fm_ref[...] = (colsum > 0.0).astype(jnp.float32)[None]

    @pl.when(b == 0)
    def _():
        r = jnp.sum(jnp.abs(bi_ref[...]))
        for l in range(L):
            r += jnp.sum(jnp.abs(wi_ref[l]))
        reg_ref[...] = jnp.full((1, 128), r, jnp.float32)


def _pool_call(h4, sp, off, cnt, fg, fb, w_inter, b_inter):
    return pl.pallas_call(
        _pool_body,
        grid=(B,),
        in_specs=[
            pl.BlockSpec((NP, D), lambda i: (0, 0)),
            pl.BlockSpec((NP, MAXF), lambda i: (0, 0)),
            pl.BlockSpec(memory_space=pltpu.SMEM),
            pl.BlockSpec(memory_space=pltpu.SMEM),
            pl.BlockSpec((1, D), lambda i: (0, 0)),
            pl.BlockSpec((1, D), lambda i: (0, 0)),
            pl.BlockSpec((L, D, D), lambda i: (0, 0, 0)),
            pl.BlockSpec((L, D), lambda i: (0, 0)),
        ],
        out_specs=[
            pl.BlockSpec((1, MAXF, D), lambda i: (i, 0, 0)),
            pl.BlockSpec((1, 1, MAXF), lambda i: (i, 0, 0)),
            pl.BlockSpec((1, 128), lambda i: (0, 0)),
        ],
        out_shape=[
            jax.ShapeDtypeStruct((B, MAXF, D), jnp.float32),
            jax.ShapeDtypeStruct((B, 1, MAXF), jnp.float32),
            jax.ShapeDtypeStruct((1, 128), jnp.float32),
        ],
    )(h4, sp, off, cnt, fg, fb, w_inter, b_inter)


def _sc_agg(t2d, gidx2, dst2):
    mesh = plsc.VectorSubcoreMesh(core_axis_name="c", subcore_axis_name="s")

    @functools.partial(
        pl.kernel,
        mesh=mesh,
        out_type=jax.ShapeDtypeStruct((2 * NP, D), jnp.float32),
        scratch_types=[
            pltpu.VMEM((GRP, CK), jnp.int32),
            pltpu.VMEM((GRP, CK), jnp.int32),
            pltpu.VMEM((CK, D), jnp.float32),
            pltpu.VMEM_SHARED((NP, D), jnp.float32),
            pltpu.SemaphoreType.DMA,
        ],
    )
    def k(t_hbm, gi_hbm, di_hbm, out_hbm, gi_g, di_g, rows, acc, sem):
        cid = lax.axis_index("c")
        sid = lax.axis_index("s")
        cnt = jnp.where(cid == 0, SPLIT0, SPLIT1)
        start = sid * CPP + cid * SPLIT0
        zero = jnp.zeros((1, 16), jnp.float32)

        @pl.loop(0, CK)
        def _(r):
            @pl.loop(0, D, step=16)
            def _(c):
                rows[pl.ds(r, 1), pl.ds(c, 16)] = zero

        @pl.loop(0, RPS // CK)
        def _(j):
            pltpu.sync_copy(rows, acc.at[pl.ds(sid * RPS + j * CK, CK)])

        plsc.subcore_barrier()

        @pl.loop(0, GMAX, step=GRP)
        def _(g0):
            pltpu.sync_copy(gi_hbm.at[pl.ds(start + g0, GRP)], gi_g)
            pltpu.sync_copy(di_hbm.at[pl.ds(start + g0, GRP)], di_g)
            for kb in range(GRP):
                @pl.when(g0 + kb < cnt)
                def _():
                    pltpu.async_copy(t_hbm.at[gi_g.at[kb]], rows, sem).wait()
                    pltpu.sync_copy(rows, acc.at[di_g.at[kb]], add=True)

        plsc.subcore_barrier()
        pltpu.sync_copy(acc.at[pl.ds(sid * RPS, RPS)],
                        out_hbm.at[pl.ds(cid * NP + sid * RPS, RPS)])

    return k(t2d, gidx2, dst2)


def kernel(x, edge_index, s, batch, mask, W_intra, b_intra, W_inter, b_inter,
           ln_gamma, ln_beta, frag_gamma, frag_beta):
    f32 = jnp.float32
    src = edge_index[0]
    dst = edge_index[1]
    gidx = src + NP * mask.astype(jnp.int32)
    gidx_p = jnp.concatenate([gidx, jnp.zeros((EP - E,), jnp.int32)]).reshape(ECH + GRP, CK)
    dst_p = jnp.concatenate([dst, jnp.full((EP - E,), N, jnp.int32)]).reshape(ECH + GRP, CK)

    x_p = jnp.pad(x, ((0, NP - N), (0, 0)))
    s_p = jnp.pad(s, ((0, NP - N), (0, 0)))
    batch_p = jnp.pad(batch, (0, NP - N), constant_values=B).reshape(NP // 128, 128)

    bsum = b_intra + b_inter

    T = _t0_call(x_p, W_inter[0], W_intra[0])
    h4 = None
    for l in range(L):
        A = _sc_agg(T.reshape(2 * NP, D), gidx_p, dst_p)
        a0 = A[:NP]
        a1 = A[NP:]
        u = T[1]
        bs = bsum[l].reshape(1, D)
        g = ln_gamma[l].reshape(1, D)
        be = ln_beta[l].reshape(1, D)
        if l < L - 1:
            T = _upd_call(u, a0, a1, bs, g, be, W_inter[l + 1], W_intra[l + 1])
        else:
            h4 = _updlast_call(u, a0, a1, bs, g, be)

    off3, cnt3 = _off_call(batch_p)
    off = off3[:, 0, 0]
    cnt = cnt3[:, 0, 0]

    fe, fm3, reg2 = _pool_call(h4, s_p, off, cnt,
                               frag_gamma.reshape(1, D), frag_beta.reshape(1, D),
                               W_inter, b_inter)
    frag_mask = fm3[:, 0, :]
    reg_loss = reg2[0, 0]
    return fe, frag_mask, reg_loss, h4[:N]


# speedup vs baseline: 1.0659x; 1.0659x over previous
import functools

import jax
import jax.numpy as jnp
from jax import lax
from jax.experimental import pallas as pl
from jax.experimental.pallas import tpu as pltpu
from jax.experimental.pallas import tpu_sc as plsc

N = 10000
E = 320000
D = 128
L = 4
B = 256
MAXF = 16
MAXN = 128
EPS = 1e-5

NP = 10240
BLK = 1024
NWORK = 32
CK = 128
ECH = 2560
GRP = 8
EP = (ECH + GRP) * CK
CPP = ECH // 16
SPLIT0 = 80
SPLIT1 = CPP - SPLIT0
GMAX = -(-max(SPLIT0, SPLIT1) // GRP) * GRP
RPS = NP // 16

_HI = jax.lax.Precision.HIGHEST


def _ln(pre, g, b):
    mu = jnp.mean(pre, axis=-1, keepdims=True)
    d = pre - mu
    var = jnp.mean(d * d, axis=-1, keepdims=True)
    return d * lax.rsqrt(var + EPS) * g + b


def _t0_body(h_ref, wi_ref, wn_ref, t_ref):
    h = h_ref[...]
    t_ref[0] = jnp.dot(h, wi_ref[...], precision=_HI)
    t_ref[1] = jnp.dot(h, wn_ref[...], precision=_HI)


def _t0_call(h, w_inter, w_intra):
    return pl.pallas_call(
        _t0_body,
        grid=(NP // BLK,),
        in_specs=[
            pl.BlockSpec((BLK, D), lambda i: (i, 0)),
            pl.BlockSpec((D, D), lambda i: (0, 0)),
            pl.BlockSpec((D, D), lambda i: (0, 0)),
        ],
        out_specs=pl.BlockSpec((2, BLK, D), lambda i: (0, i, 0)),
        out_shape=jax.ShapeDtypeStruct((2, NP, D), jnp.float32),
    )(h, w_inter, w_intra)


def _upd_body(u_ref, a0_ref, a1_ref, bs_ref, g_ref, b_ref, wi_ref, wn_ref, t_ref):
    pre = u_ref[...] + a0_ref[...] + a1_ref[...] + bs_ref[...]
    h = jnp.maximum(_ln(pre, g_ref[...], b_ref[...]), 0.0)
    t_ref[0] = jnp.dot(h, wi_ref[...], precision=_HI)
    t_ref[1] = jnp.dot(h, wn_ref[...], precision=_HI)


def _upd_call(u, a0, a1, bs, g, b, w_inter, w_intra):
    return pl.pallas_call(
        _upd_body,
        grid=(NP // BLK,),
        in_specs=[
            pl.BlockSpec((BLK, D), lambda i: (i, 0)),
            pl.BlockSpec((BLK, D), lambda i: (i, 0)),
            pl.BlockSpec((BLK, D), lambda i: (i, 0)),
            pl.BlockSpec((1, D), lambda i: (0, 0)),
            pl.BlockSpec((1, D), lambda i: (0, 0)),
            pl.BlockSpec((1, D), lambda i: (0, 0)),
            pl.BlockSpec((D, D), lambda i: (0, 0)),
            pl.BlockSpec((D, D), lambda i: (0, 0)),
        ],
        out_specs=pl.BlockSpec((2, BLK, D), lambda i: (0, i, 0)),
        out_shape=jax.ShapeDtypeStruct((2, NP, D), jnp.float32),
    )(u, a0, a1, bs, g, b, w_inter, w_intra)


def _updlast_body(u_ref, a0_ref, a1_ref, bs_ref, g_ref, b_ref, h_ref):
    pre = u_ref[...] + a0_ref[...] + a1_ref[...] + bs_ref[...]
    h_ref[...] = jnp.maximum(_ln(pre, g_ref[...], b_ref[...]), 0.0)


def _updlast_call(u, a0, a1, bs, g, b):
    return pl.pallas_call(
        _updlast_body,
        grid=(NP // BLK,),
        in_specs=[
            pl.BlockSpec((BLK, D), lambda i: (i, 0)),
            pl.BlockSpec((BLK, D), lambda i: (i, 0)),
            pl.BlockSpec((BLK, D), lambda i: (i, 0)),
            pl.BlockSpec((1, D), lambda i: (0, 0)),
            pl.BlockSpec((1, D), lambda i: (0, 0)),
            pl.BlockSpec((1, D), lambda i: (0, 0)),
        ],
        out_specs=pl.BlockSpec((BLK, D), lambda i: (i, 0)),
        out_shape=jax.ShapeDtypeStruct((NP, D), jnp.float32),
    )(u, a0, a1, bs, g, b)


def _off_body(bv_ref, off_ref, cnt_ref):
    b = pl.program_id(0)
    bv = bv_ref[...]
    off = jnp.sum((bv < b).astype(jnp.int32))
    cnt = jnp.sum((bv == b).astype(jnp.int32))
    off_ref[...] = jnp.full((1, 1, 128), off, jnp.int32)
    cnt_ref[...] = jnp.full((1, 1, 128), cnt, jnp.int32)


def _off_call(bv):
    return pl.pallas_call(
        _off_body,
        grid=(B,),
        in_specs=[pl.BlockSpec((NP // 128, 128), lambda i: (0, 0))],
        out_specs=[
            pl.BlockSpec((1, 1, 128), lambda i: (i, 0, 0)),
            pl.BlockSpec((1, 1, 128), lambda i: (i, 0, 0)),
        ],
        out_shape=[
            jax.ShapeDtypeStruct((B, 1, 128), jnp.int32),
            jax.ShapeDtypeStruct((B, 1, 128), jnp.int32),
        ],
    )(bv)


def _pool_body(h_ref, s_ref, off_ref, cnt_ref, fg_ref, fb_ref, wi_ref, bi_ref,
               fe_ref, fm_ref, reg_ref):
    b = pl.program_id(0)
    off = off_ref[b]
    cnt = cnt_ref[b]
    hs = h_ref[pl.ds(off, MAXN), :]
    ss = s_ref[pl.ds(off, MAXN), :]
    m = (lax.broadcasted_iota(jnp.int32, (MAXN, 1), 0) < cnt).astype(jnp.float32)
    ssm = ss * m
    fe = lax.dot_general(ssm, hs, (((0,), (0,)), ((), ())),
                         precision=_HI, preferred_element_type=jnp.float32)
    fe_ref[...] = _ln(fe, fg_ref[...], fb_ref[...])[None]
    colsum = jnp.sum(ssm, axis=0, keepdims=True)
    fm_ref[...] = (colsum > 0.0).astype(jnp.float32)[None]

    @pl.when(b == 0)
    def _():
        r = jnp.sum(jnp.abs(bi_ref[...]))
        for l in range(L):
            r += jnp.sum(jnp.abs(wi_ref[l]))
        reg_ref[...] = jnp.full((1, 128), r, jnp.float32)


def _pool_call(h4, sp, off, cnt, fg, fb, w_inter, b_inter):
    return pl.pallas_call(
        _pool_body,
        grid=(B,),
        in_specs=[
            pl.BlockSpec((NP, D), lambda i: (0, 0)),
            pl.BlockSpec((NP, MAXF), lambda i: (0, 0)),
            pl.BlockSpec(memory_space=pltpu.SMEM),
            pl.BlockSpec(memory_space=pltpu.SMEM),
            pl.BlockSpec((1, D), lambda i: (0, 0)),
            pl.BlockSpec((1, D), lambda i: (0, 0)),
            pl.BlockSpec((L, D, D), lambda i: (0, 0, 0)),
            pl.BlockSpec((L, D), lambda i: (0, 0)),
        ],
        out_specs=[
            pl.BlockSpec((1, MAXF, D), lambda i: (i, 0, 0)),
            pl.BlockSpec((1, 1, MAXF), lambda i: (i, 0, 0)),
            pl.BlockSpec((1, 128), lambda i: (0, 0)),
        ],
        out_shape=[
            jax.ShapeDtypeStruct((B, MAXF, D), jnp.float32),
            jax.ShapeDtypeStruct((B, 1, MAXF), jnp.float32),
            jax.ShapeDtypeStruct((1, 128), jnp.float32),
        ],
    )(h4, sp, off, cnt, fg, fb, w_inter, b_inter)


def _sc_agg(t2d, gidx2, dst2):
    mesh = plsc.VectorSubcoreMesh(core_axis_name="c", subcore_axis_name="s")

    @functools.partial(
        pl.kernel,
        mesh=mesh,
        out_type=jax.ShapeDtypeStruct((2 * NP, D), jnp.float32),
        scratch_types=[
            pltpu.VMEM((GRP, CK), jnp.int32),
            pltpu.VMEM((GRP, CK), jnp.int32),
            pltpu.VMEM((CK, D), jnp.float32),
            pltpu.VMEM_SHARED((NP, D), jnp.float32),
            pltpu.SemaphoreType.DMA,
        ],
    )
    def k(t_hbm, gi_hbm, di_hbm, out_hbm, gi_g, di_g, rows, acc, sem):
        cid = lax.axis_index("c")
        sid = lax.axis_index("s")
        cnt = jnp.where(cid == 0, SPLIT0, SPLIT1)
        start = sid * CPP + cid * SPLIT0
        zero = jnp.zeros((1, 16), jnp.float32)

        @pl.loop(0, CK)
        def _(r):
            @pl.loop(0, D, step=16)
            def _(c):
                rows[pl.ds(r, 1), pl.ds(c, 16)] = zero

        @pl.loop(0, RPS // CK)
        def _(j):
            pltpu.sync_copy(rows, acc.at[pl.ds(sid * RPS + j * CK, CK)])

        plsc.subcore_barrier()

        @pl.loop(0, GMAX, step=GRP)
        def _(g0):
            pltpu.sync_copy(gi_hbm.at[pl.ds(start + g0, GRP)], gi_g)
            pltpu.sync_copy(di_hbm.at[pl.ds(start + g0, GRP)], di_g)
            for kb in range(GRP):
                @pl.when(g0 + kb < cnt)
                def _():
                    pltpu.async_copy(t_hbm.at[gi_g.at[kb]], rows, sem).wait()
                    pltpu.sync_copy(rows, acc.at[di_g.at[kb]], add=True)

        plsc.subcore_barrier()
        pltpu.sync_copy(acc.at[pl.ds(sid * RPS, RPS)],
                        out_hbm.at[pl.ds(cid * NP + sid * RPS, RPS)])

    return k(t2d, gidx2, dst2)


def kernel(x, edge_index, s, batch, mask, W_intra, b_intra, W_inter, b_inter,
           ln_gamma, ln_beta, frag_gamma, frag_beta):
    f32 = jnp.float32
    src = edge_index[0]
    dst = edge_index[1]
    gidx = src + NP * mask.astype(jnp.int32)
    gidx_p = jnp.concatenate([gidx, jnp.zeros((EP - E,), jnp.int32)]).reshape(ECH + GRP, CK)
    dst_p = jnp.concatenate([dst, jnp.full((EP - E,), N, jnp.int32)]).reshape(ECH + GRP, CK)

    x_p = jnp.pad(x, ((0, NP - N), (0, 0)))
    s_p = jnp.pad(s, ((0, NP - N), (0, 0)))
    batch_p = jnp.pad(batch, (0, NP - N), constant_values=B).reshape(NP // 128, 128)

    bsum = b_intra + b_inter

    T = _t0_call(x_p, W_inter[0], W_intra[0])
    h4 = None
    for l in range(L):
        A = _sc_agg(T.reshape(2 * NP, D), gidx_p, dst_p)
        a0 = A[:NP]
        a1 = A[NP:]
        u = T[1]
        bs = bsum[l].reshape(1, D)
        g = ln_gamma[l].reshape(1, D)
        be = ln_beta[l].reshape(1, D)
        if l < L - 1:
            T = _upd_call(u, a0, a1, bs, g, be, W_inter[l + 1], W_intra[l + 1])
        else:
            h4 = _updlast_call(u, a0, a1, bs, g, be)

    off3, cnt3 = _off_call(batch_p)
    off = off3[:, 0, 0]
    cnt = cnt3[:, 0, 0]

    fe, fm3, reg2 = _pool_call(h4, s_p, off, cnt,
                               frag_gamma.reshape(1, D), frag_beta.reshape(1, D),
                               W_inter, b_inter)
    frag_mask = fm3[:, 0, :]
    reg_loss = reg2[0, 0]
    return fe, frag_mask, reg_loss, h4[:N]


# speedup vs baseline: 1.2648x; 1.1866x over previous
import functools

import jax
import jax.numpy as jnp
from jax import lax
from jax.experimental import pallas as pl
from jax.experimental.pallas import tpu as pltpu
from jax.experimental.pallas import tpu_sc as plsc

N = 10000
E = 320000
D = 128
L = 4
B = 256
MAXF = 16
MAXN = 128
EPS = 1e-5

NP = 10240
BLK = 1024
NWORK = 32
CK = 128
ECH = 2528
EP = ECH * CK
CPW = ECH // NWORK
RPS = NP // 16

_HI = jax.lax.Precision.HIGHEST


def _ln(pre, g, b):
    mu = jnp.mean(pre, axis=-1, keepdims=True)
    d = pre - mu
    var = jnp.mean(d * d, axis=-1, keepdims=True)
    return d * lax.rsqrt(var + EPS) * g + b


def _t0_body(h_ref, wi_ref, wn_ref, t_ref):
    h = h_ref[...]
    t_ref[0] = jnp.dot(h, wi_ref[...], precision=_HI)
    t_ref[1] = jnp.dot(h, wn_ref[...], precision=_HI)


def _t0_call(h, w_inter, w_intra):
    return pl.pallas_call(
        _t0_body,
        grid=(NP // BLK,),
        in_specs=[
            pl.BlockSpec((BLK, D), lambda i: (i, 0)),
            pl.BlockSpec((D, D), lambda i: (0, 0)),
            pl.BlockSpec((D, D), lambda i: (0, 0)),
        ],
        out_specs=pl.BlockSpec((2, BLK, D), lambda i: (0, i, 0)),
        out_shape=jax.ShapeDtypeStruct((2, NP, D), jnp.float32),
    )(h, w_inter, w_intra)


def _upd_body(u_ref, a0_ref, a1_ref, bs_ref, g_ref, b_ref, wi_ref, wn_ref, t_ref):
    pre = u_ref[...] + a0_ref[...] + a1_ref[...] + bs_ref[...]
    h = jnp.maximum(_ln(pre, g_ref[...], b_ref[...]), 0.0)
    t_ref[0] = jnp.dot(h, wi_ref[...], precision=_HI)
    t_ref[1] = jnp.dot(h, wn_ref[...], precision=_HI)


def _upd_call(u, a0, a1, bs, g, b, w_inter, w_intra):
    return pl.pallas_call(
        _upd_body,
        grid=(NP // BLK,),
        in_specs=[
            pl.BlockSpec((BLK, D), lambda i: (i, 0)),
            pl.BlockSpec((BLK, D), lambda i: (i, 0)),
            pl.BlockSpec((BLK, D), lambda i: (i, 0)),
            pl.BlockSpec((1, D), lambda i: (0, 0)),
            pl.BlockSpec((1, D), lambda i: (0, 0)),
            pl.BlockSpec((1, D), lambda i: (0, 0)),
            pl.BlockSpec((D, D), lambda i: (0, 0)),
            pl.BlockSpec((D, D), lambda i: (0, 0)),
        ],
        out_specs=pl.BlockSpec((2, BLK, D), lambda i: (0, i, 0)),
        out_shape=jax.ShapeDtypeStruct((2, NP, D), jnp.float32),
    )(u, a0, a1, bs, g, b, w_inter, w_intra)


def _updlast_body(u_ref, a0_ref, a1_ref, bs_ref, g_ref, b_ref, h_ref):
    pre = u_ref[...] + a0_ref[...] + a1_ref[...] + bs_ref[...]
    h_ref[...] = jnp.maximum(_ln(pre, g_ref[...], b_ref[...]), 0.0)


def _updlast_call(u, a0, a1, bs, g, b):
    return pl.pallas_call(
        _updlast_body,
        grid=(NP // BLK,),
        in_specs=[
            pl.BlockSpec((BLK, D), lambda i: (i, 0)),
            pl.BlockSpec((BLK, D), lambda i: (i, 0)),
            pl.BlockSpec((BLK, D), lambda i: (i, 0)),
            pl.BlockSpec((1, D), lambda i: (0, 0)),
            pl.BlockSpec((1, D), lambda i: (0, 0)),
            pl.BlockSpec((1, D), lambda i: (0, 0)),
        ],
        out_specs=pl.BlockSpec((BLK, D), lambda i: (i, 0)),
        out_shape=jax.ShapeDtypeStruct((NP, D), jnp.float32),
    )(u, a0, a1, bs, g, b)


def _off_body(bv_ref, off_ref, cnt_ref):
    b = pl.program_id(0)
    bv = bv_ref[...]
    off = jnp.sum((bv < b).astype(jnp.int32))
    cnt = jnp.sum((bv == b).astype(jnp.int32))
    off_ref[...] = jnp.full((1, 1, 128), off, jnp.int32)
    cnt_ref[...] = jnp.full((1, 1, 128), cnt, jnp.int32)


def _off_call(bv):
    return pl.pallas_call(
        _off_body,
        grid=(B,),
        in_specs=[pl.BlockSpec((NP // 128, 128), lambda i: (0, 0))],
        out_specs=[
            pl.BlockSpec((1, 1, 128), lambda i: (i, 0, 0)),
            pl.BlockSpec((1, 1, 128), lambda i: (i, 0, 0)),
        ],
        out_shape=[
            jax.ShapeDtypeStruct((B, 1, 128), jnp.int32),
            jax.ShapeDtypeStruct((B, 1, 128), jnp.int32),
        ],
    )(bv)


def _pool_body(h_ref, s_ref, off_ref, cnt_ref, fg_ref, fb_ref, wi_ref, bi_ref,
               fe_ref, fm_ref, reg_ref):
    b = pl.program_id(0)
    off = off_ref[b]
    cnt = cnt_ref[b]
    hs = h_ref[pl.ds(off, MAXN), :]
    ss = s_ref[pl.ds(off, MAXN), :]
    m = (lax.broadcasted_iota(jnp.int32, (MAXN, 1), 0) < cnt).astype(jnp.float32)
    ssm = ss * m
    fe = lax.dot_general(ssm, hs, (((0,), (0,)), ((), ())),
                         precision=_HI, preferred_element_type=jnp.float32)
    fe_ref[...] = _ln(fe, fg_ref[...], fb_ref[...])[None]
    colsum = jnp.sum(ssm, axis=0, keepdims=True)
    fm_ref[...] = (colsum > 0.0).astype(jnp.float32)[None]

    @pl.when(b == 0)
    def _():
        r = jnp.sum(jnp.abs(bi_ref[...]))
        for l in range(L):
            r += jnp.sum(jnp.abs(wi_ref[l]))
        reg_ref[...] = jnp.full((1, 128), r, jnp.float32)


def _pool_call(h4, sp, off, cnt, fg, fb, w_inter, b_inter):
    return pl.pallas_call(
        _pool_body,
        grid=(B,),
        in_specs=[
            pl.BlockSpec((NP, D), lambda i: (0, 0)),
            pl.BlockSpec((NP, MAXF), lambda i: (0, 0)),
            pl.BlockSpec(memory_space=pltpu.SMEM),
            pl.BlockSpec(memory_space=pltpu.SMEM),
            pl.BlockSpec((1, D), lambda i: (0, 0)),
            pl.BlockSpec((1, D), lambda i: (0, 0)),
            pl.BlockSpec((L, D, D), lambda i: (0, 0, 0)),
            pl.BlockSpec((L, D), lambda i: (0, 0)),
        ],
        out_specs=[
            pl.BlockSpec((1, MAXF, D), lambda i: (i, 0, 0)),
            pl.BlockSpec((1, 1, MAXF), lambda i: (i, 0, 0)),
            pl.BlockSpec((1, 128), lambda i: (0, 0)),
        ],
        out_shape=[
            jax.ShapeDtypeStruct((B, MAXF, D), jnp.float32),
            jax.ShapeDtypeStruct((B, 1, MAXF), jnp.float32),
            jax.ShapeDtypeStruct((1, 128), jnp.float32),
        ],
    )(h4, sp, off, cnt, fg, fb, w_inter, b_inter)


def _sc_agg(t2d, gidx, dstp):
    mesh = plsc.VectorSubcoreMesh(core_axis_name="c", subcore_axis_name="s")

    @functools.partial(
        pl.kernel,
        mesh=mesh,
        out_type=jax.ShapeDtypeStruct((2 * NP, D), jnp.float32),
        scratch_types=[
            pltpu.VMEM((CK,), jnp.int32),
            pltpu.VMEM((CK,), jnp.int32),
            pltpu.VMEM((CK, D), jnp.float32),
            pltpu.VMEM_SHARED((NP, D), jnp.float32),
            pltpu.SemaphoreType.DMA,
        ],
    )
    def k(t_hbm, gi_hbm, di_hbm, out_hbm, idx_v, dst_v, rows_v, acc, sem):
        cid = lax.axis_index("c")
        sid = lax.axis_index("s")
        wid = sid * 2 + cid
        zero = jnp.zeros((1, 16), jnp.float32)

        @pl.loop(0, CK)
        def _(r):
            @pl.loop(0, D, step=16)
            def _(c):
                rows_v[pl.ds(r, 1), pl.ds(c, 16)] = zero

        @pl.loop(0, RPS // CK)
        def _(j):
            pltpu.sync_copy(rows_v, acc.at[pl.ds(sid * RPS + j * CK, CK)])

        plsc.subcore_barrier()

        @pl.loop(0, CPW)
        def _(j):
            base = (wid * CPW + j) * CK
            pltpu.sync_copy(gi_hbm.at[pl.ds(base, CK)], idx_v)
            pltpu.sync_copy(di_hbm.at[pl.ds(base, CK)], dst_v)
            pltpu.async_copy(t_hbm.at[idx_v], rows_v, sem).wait()
            pltpu.sync_copy(rows_v, acc.at[dst_v], add=True)

        plsc.subcore_barrier()
        pltpu.sync_copy(acc.at[pl.ds(sid * RPS, RPS)],
                        out_hbm.at[pl.ds(cid * NP + sid * RPS, RPS)])

    return k(t2d, gidx, dstp)


def kernel(x, edge_index, s, batch, mask, W_intra, b_intra, W_inter, b_inter,
           ln_gamma, ln_beta, frag_gamma, frag_beta):
    f32 = jnp.float32
    src = edge_index[0]
    dst = edge_index[1]
    gidx = src + NP * mask.astype(jnp.int32)
    gidx_p = jnp.concatenate([gidx, jnp.zeros((EP - E,), jnp.int32)])
    dst_p = jnp.concatenate([dst, jnp.full((EP - E,), N, jnp.int32)])

    x_p = jnp.pad(x, ((0, NP - N), (0, 0)))
    s_p = jnp.pad(s, ((0, NP - N), (0, 0)))
    batch_p = jnp.pad(batch, (0, NP - N), constant_values=B).reshape(NP // 128, 128)

    bsum = b_intra + b_inter

    T = _t0_call(x_p, W_inter[0], W_intra[0])
    h4 = None
    for l in range(L):
        A = _sc_agg(T.reshape(2 * NP, D), gidx_p, dst_p)
        a0 = A[:NP]
        a1 = A[NP:]
        u = T[1]
        bs = bsum[l].reshape(1, D)
        g = ln_gamma[l].reshape(1, D)
        be = ln_beta[l].reshape(1, D)
        if l < L - 1:
            T = _upd_call(u, a0, a1, bs, g, be, W_inter[l + 1], W_intra[l + 1])
        else:
            h4 = _updlast_call(u, a0, a1, bs, g, be)

    off3, cnt3 = _off_call(batch_p)
    off = off3[:, 0, 0]
    cnt = cnt3[:, 0, 0]

    fe, fm3, reg2 = _pool_call(h4, s_p, off, cnt,
                               frag_gamma.reshape(1, D), frag_beta.reshape(1, D),
                               W_inter, b_inter)
    frag_mask = fm3[:, 0, :]
    reg_loss = reg2[0, 0]
    return fe, frag_mask, reg_loss, h4[:N]


# speedup vs baseline: 1.4213x; 1.1237x over previous
import functools

import jax
import jax.numpy as jnp
from jax import lax
from jax.experimental import pallas as pl
from jax.experimental.pallas import tpu as pltpu
from jax.experimental.pallas import tpu_sc as plsc

N = 10000
E = 320000
D = 128
L = 4
B = 256
MAXF = 16
MAXN = 128
EPS = 1e-5

NP = 10240
BLK = 1024
NWORK = 32
CK = 128
ECH = 2528
EP = ECH * CK
CPW = ECH // NWORK
RPS = NP // 16

_HI = jax.lax.Precision.HIGHEST


def _ln(pre, g, b):
    mu = jnp.mean(pre, axis=-1, keepdims=True)
    d = pre - mu
    var = jnp.mean(d * d, axis=-1, keepdims=True)
    return d * lax.rsqrt(var + EPS) * g + b


def _t0_body(h_ref, wi_ref, wn_ref, t_ref):
    h = h_ref[...]
    t_ref[0] = jnp.dot(h, wi_ref[...], precision=_HI)
    t_ref[1] = jnp.dot(h, wn_ref[...], precision=_HI)


def _t0_call(h, w_inter, w_intra):
    return pl.pallas_call(
        _t0_body,
        grid=(NP // BLK,),
        in_specs=[
            pl.BlockSpec((BLK, D), lambda i: (i, 0)),
            pl.BlockSpec((D, D), lambda i: (0, 0)),
            pl.BlockSpec((D, D), lambda i: (0, 0)),
        ],
        out_specs=pl.BlockSpec((2, BLK, D), lambda i: (0, i, 0)),
        out_shape=jax.ShapeDtypeStruct((2, NP, D), jnp.float32),
    )(h, w_inter, w_intra)


def _upd_body(u_ref, a0_ref, a1_ref, bs_ref, g_ref, b_ref, wi_ref, wn_ref, t_ref):
    pre = u_ref[...] + a0_ref[...] + a1_ref[...] + bs_ref[...]
    h = jnp.maximum(_ln(pre, g_ref[...], b_ref[...]), 0.0)
    t_ref[0] = jnp.dot(h, wi_ref[...], precision=_HI)
    t_ref[1] = jnp.dot(h, wn_ref[...], precision=_HI)


def _upd_call(u, a0, a1, bs, g, b, w_inter, w_intra):
    return pl.pallas_call(
        _upd_body,
        grid=(NP // BLK,),
        in_specs=[
            pl.BlockSpec((BLK, D), lambda i: (i, 0)),
            pl.BlockSpec((BLK, D), lambda i: (i, 0)),
            pl.BlockSpec((BLK, D), lambda i: (i, 0)),
            pl.BlockSpec((1, D), lambda i: (0, 0)),
            pl.BlockSpec((1, D), lambda i: (0, 0)),
            pl.BlockSpec((1, D), lambda i: (0, 0)),
            pl.BlockSpec((D, D), lambda i: (0, 0)),
            pl.BlockSpec((D, D), lambda i: (0, 0)),
        ],
        out_specs=pl.BlockSpec((2, BLK, D), lambda i: (0, i, 0)),
        out_shape=jax.ShapeDtypeStruct((2, NP, D), jnp.float32),
    )(u, a0, a1, bs, g, b, w_inter, w_intra)


def _updlast_body(u_ref, a0_ref, a1_ref, bs_ref, g_ref, b_ref, h_ref):
    pre = u_ref[...] + a0_ref[...] + a1_ref[...] + bs_ref[...]
    h_ref[...] = jnp.maximum(_ln(pre, g_ref[...], b_ref[...]), 0.0)


def _updlast_call(u, a0, a1, bs, g, b):
    return pl.pallas_call(
        _updlast_body,
        grid=(NP // BLK,),
        in_specs=[
            pl.BlockSpec((BLK, D), lambda i: (i, 0)),
            pl.BlockSpec((BLK, D), lambda i: (i, 0)),
            pl.BlockSpec((BLK, D), lambda i: (i, 0)),
            pl.BlockSpec((1, D), lambda i: (0, 0)),
            pl.BlockSpec((1, D), lambda i: (0, 0)),
            pl.BlockSpec((1, D), lambda i: (0, 0)),
        ],
        out_specs=pl.BlockSpec((BLK, D), lambda i: (i, 0)),
        out_shape=jax.ShapeDtypeStruct((NP, D), jnp.float32),
    )(u, a0, a1, bs, g, b)


def _off_body(bv_ref, off_ref, cnt_ref):
    b = pl.program_id(0)
    bv = bv_ref[...]
    off = jnp.sum((bv < b).astype(jnp.int32))
    cnt = jnp.sum((bv == b).astype(jnp.int32))
    off_ref[...] = jnp.full((1, 1, 128), off, jnp.int32)
    cnt_ref[...] = jnp.full((1, 1, 128), cnt, jnp.int32)


def _off_call(bv):
    return pl.pallas_call(
        _off_body,
        grid=(B,),
        in_specs=[pl.BlockSpec((NP // 128, 128), lambda i: (0, 0))],
        out_specs=[
            pl.BlockSpec((1, 1, 128), lambda i: (i, 0, 0)),
            pl.BlockSpec((1, 1, 128), lambda i: (i, 0, 0)),
        ],
        out_shape=[
            jax.ShapeDtypeStruct((B, 1, 128), jnp.int32),
            jax.ShapeDtypeStruct((B, 1, 128), jnp.int32),
        ],
    )(bv)


def _pool_body(h_ref, s_ref, off_ref, cnt_ref, fg_ref, fb_ref, wi_ref, bi_ref,
               fe_ref, fm_ref, reg_ref):
    b = pl.program_id(0)
    off = off_ref[b]
    cnt = cnt_ref[b]
    hs = h_ref[pl.ds(off, MAXN), :]
    ss = s_ref[pl.ds(off, MAXN), :]
    m = (lax.broadcasted_iota(jnp.int32, (MAXN, 1), 0) < cnt).astype(jnp.float32)
    ssm = ss * m
    fe = lax.dot_general(ssm, hs, (((0,), (0,)), ((), ())),
                         precision=_HI, preferred_element_type=jnp.float32)
    fe_ref[...] = _ln(fe, fg_ref[...], fb_ref[...])[None]
    colsum = jnp.sum(ssm, axis=0, keepdims=True)
    fm_ref[...] = (colsum > 0.0).astype(jnp.float32)[None]

    @pl.when(b == 0)
    def _():
        r = jnp.sum(jnp.abs(bi_ref[...]))
        for l in range(L):
            r += jnp.sum(jnp.abs(wi_ref[l]))
        reg_ref[...] = jnp.full((1, 128), r, jnp.float32)


def _pool_call(h4, sp, off, cnt, fg, fb, w_inter, b_inter):
    return pl.pallas_call(
        _pool_body,
        grid=(B,),
        in_specs=[
            pl.BlockSpec((NP, D), lambda i: (0, 0)),
            pl.BlockSpec((NP, MAXF), lambda i: (0, 0)),
            pl.BlockSpec(memory_space=pltpu.SMEM),
            pl.BlockSpec(memory_space=pltpu.SMEM),
            pl.BlockSpec((1, D), lambda i: (0, 0)),
            pl.BlockSpec((1, D), lambda i: (0, 0)),
            pl.BlockSpec((L, D, D), lambda i: (0, 0, 0)),
            pl.BlockSpec((L, D), lambda i: (0, 0)),
        ],
        out_specs=[
            pl.BlockSpec((1, MAXF, D), lambda i: (i, 0, 0)),
            pl.BlockSpec((1, 1, MAXF), lambda i: (i, 0, 0)),
            pl.BlockSpec((1, 128), lambda i: (0, 0)),
        ],
        out_shape=[
            jax.ShapeDtypeStruct((B, MAXF, D), jnp.float32),
            jax.ShapeDtypeStruct((B, 1, MAXF), jnp.float32),
            jax.ShapeDtypeStruct((1, 128), jnp.float32),
        ],
    )(h4, sp, off, cnt, fg, fb, w_inter, b_inter)


def _sc_agg(t2d, ed):
    mesh = plsc.VectorSubcoreMesh(core_axis_name="c", subcore_axis_name="s")

    @functools.partial(
        pl.kernel,
        mesh=mesh,
        out_type=jax.ShapeDtypeStruct((2 * NP, D), jnp.float32),
        scratch_types=[
            pltpu.VMEM((2, CK), jnp.int32),
            pltpu.VMEM((CK, D), jnp.float32),
            pltpu.VMEM_SHARED((NP, D), jnp.float32),
            pltpu.SemaphoreType.DMA,
        ],
    )
    def k(t_hbm, ed_hbm, out_hbm, ed_v, rows_v, acc, sem):
        cid = lax.axis_index("c")
        sid = lax.axis_index("s")
        wid = sid * 2 + cid
        zero = jnp.zeros((1, 16), jnp.float32)

        @pl.loop(0, CK)
        def _(r):
            @pl.loop(0, D, step=16)
            def _(c):
                rows_v[pl.ds(r, 1), pl.ds(c, 16)] = zero

        @pl.loop(0, RPS // CK)
        def _(j):
            pltpu.sync_copy(rows_v, acc.at[pl.ds(sid * RPS + j * CK, CK)])

        plsc.subcore_barrier()

        @pl.loop(0, CPW)
        def _(j):
            pltpu.sync_copy(ed_hbm.at[wid * CPW + j], ed_v)
            pltpu.async_copy(t_hbm.at[ed_v.at[0]], rows_v, sem).wait()
            pltpu.sync_copy(rows_v, acc.at[ed_v.at[1]], add=True)

        plsc.subcore_barrier()
        pltpu.sync_copy(acc.at[pl.ds(sid * RPS, RPS)],
                        out_hbm.at[pl.ds(cid * NP + sid * RPS, RPS)])

    return k(t2d, ed)


def kernel(x, edge_index, s, batch, mask, W_intra, b_intra, W_inter, b_inter,
           ln_gamma, ln_beta, frag_gamma, frag_beta):
    f32 = jnp.float32
    src = edge_index[0]
    dst = edge_index[1]
    gidx = src + NP * mask.astype(jnp.int32)
    gidx_p = jnp.concatenate([gidx, jnp.zeros((EP - E,), jnp.int32)]).reshape(ECH, 1, CK)
    dst_p = jnp.concatenate([dst, jnp.full((EP - E,), N, jnp.int32)]).reshape(ECH, 1, CK)
    ed_p = jnp.concatenate([gidx_p, dst_p], axis=1)

    x_p = jnp.pad(x, ((0, NP - N), (0, 0)))
    s_p = jnp.pad(s, ((0, NP - N), (0, 0)))
    batch_p = jnp.pad(batch, (0, NP - N), constant_values=B).reshape(NP // 128, 128)

    bsum = b_intra + b_inter

    T = _t0_call(x_p, W_inter[0], W_intra[0])
    h4 = None
    for l in range(L):
        A = _sc_agg(T.reshape(2 * NP, D), ed_p)
        a0 = A[:NP]
        a1 = A[NP:]
        u = T[1]
        bs = bsum[l].reshape(1, D)
        g = ln_gamma[l].reshape(1, D)
        be = ln_beta[l].reshape(1, D)
        if l < L - 1:
            T = _upd_call(u, a0, a1, bs, g, be, W_inter[l + 1], W_intra[l + 1])
        else:
            h4 = _updlast_call(u, a0, a1, bs, g, be)

    off3, cnt3 = _off_call(batch_p)
    off = off3[:, 0, 0]
    cnt = cnt3[:, 0, 0]

    fe, fm3, reg2 = _pool_call(h4, s_p, off, cnt,
                               frag_gamma.reshape(1, D), frag_beta.reshape(1, D),
                               W_inter, b_inter)
    frag_mask = fm3[:, 0, :]
    reg_loss = reg2[0, 0]
    return fe, frag_mask, reg_loss, h4[:N]
